# baseline (speedup 1.0000x reference)
import jax
import jax.numpy as jnp
from jax.experimental import pallas as pl
from jax.experimental.pallas import tpu as pltpu

LAMBDA_COE = 0.5
EPS = 1e-12

D = 1024
N = 8192
B = 8192
BN = 1024
NBLK = N // BN
CW = 512
NCH = BN // CW


def _body(ce_ref, h_ref, out_ref, s_acc, tr_acc):
    i = pl.program_id(0)

    @pl.when(i == 0)
    def _init():
        s_acc[...] = jnp.zeros_like(s_acc)
        tr_acc[...] = jnp.zeros_like(tr_acc)

    ones_row = jnp.ones((1, D), dtype=jnp.bfloat16)
    for c in range(NCH):
        hb = h_ref[:, c * CW:(c + 1) * CW].astype(jnp.bfloat16)
        hsq = hb * hb
        colnorm2 = jax.lax.dot_general(
            ones_row, hsq, (((1,), (0,)), ((), ())),
            preferred_element_type=jnp.float32)
        inv = jax.lax.rsqrt(jnp.maximum(colnorm2, EPS * EPS))
        tr_acc[:, c * CW:(c + 1) * CW] += colnorm2 * inv * inv
        s_acc[...] += jax.lax.dot_general(
            inv.astype(jnp.bfloat16), hb, (((1,), (1,)), ((), ())),
            preferred_element_type=jnp.float32)

    @pl.when(i == NBLK - 1)
    def _finalize():
        s = s_acc[...]
        sum_g = jnp.sum(s * s)
        pair_sum = (sum_g - jnp.sum(tr_acc[...])) * 0.5
        reg = pair_sum * LAMBDA_COE / (N * (N - 1) / 2)

        o0 = ce_ref[0]
        o1 = ce_ref[1]
        labf = ce_ref[2]
        m = jnp.maximum(o0, o1)
        lse = m + jnp.log(jnp.exp(o0 - m) + jnp.exp(o1 - m))
        chosen = jnp.where(labf == 1.0, o1, o0)
        ce = jnp.sum(lse - chosen) / B

        out_ref[...] = jnp.reshape(ce + reg, (1, 1))


def kernel(outputs, labels, H):
    packed = jnp.stack([
        outputs[:, 0].reshape(64, 128),
        outputs[:, 1].reshape(64, 128),
        labels.astype(jnp.float32).reshape(64, 128),
    ])

    out = pl.pallas_call(
        _body,
        grid=(NBLK,),
        in_specs=[
            pl.BlockSpec((3, 64, 128), lambda i: (0, 0, 0)),
            pl.BlockSpec((D, BN), lambda i: (0, i)),
        ],
        out_specs=pl.BlockSpec((1, 1), lambda i: (0, 0)),
        out_shape=jax.ShapeDtypeStruct((1, 1), jnp.float32),
        scratch_shapes=[
            pltpu.VMEM((1, D), jnp.float32),
            pltpu.VMEM((1, BN), jnp.float32),
        ],
        compiler_params=pltpu.CompilerParams(
            dimension_semantics=("arbitrary",),
        ),
    )(packed, H)
    return out[0, 0]


# speedup vs baseline: 1.1659x; 1.1659x over previous
import jax
import jax.numpy as jnp
from jax.experimental import pallas as pl
from jax.experimental.pallas import tpu as pltpu

LAMBDA_COE = 0.5
EPS = 1e-12

D = 1024
N = 8192
B = 8192
BN = 2048
NBLK = N // BN


def _body(o0_ref, o1_ref, lab_ref, h_ref, out_ref, s_acc, tr_acc):
    i = pl.program_id(0)

    @pl.when(i == 0)
    def _init():
        s_acc[...] = jnp.zeros_like(s_acc)
        tr_acc[...] = jnp.zeros_like(tr_acc)

    h = h_ref[...]
    hb = h.astype(jnp.bfloat16)
    hsq = hb * hb
    ones_row = jnp.ones((1, D), dtype=jnp.bfloat16)
    colnorm2 = jax.lax.dot_general(
        ones_row, hsq, (((1,), (0,)), ((), ())),
        preferred_element_type=jnp.float32)
    inv = jax.lax.rsqrt(jnp.maximum(colnorm2, EPS * EPS))
    tr_acc[...] += colnorm2 * inv * inv
    s_acc[...] += jax.lax.dot_general(
        inv.astype(jnp.bfloat16), hb, (((1,), (1,)), ((), ())),
        preferred_element_type=jnp.float32)

    @pl.when(i == NBLK - 1)
    def _finalize():
        s = s_acc[...]
        sum_g = jnp.sum(s * s)
        pair_sum = (sum_g - jnp.sum(tr_acc[...])) * 0.5
        reg = pair_sum * LAMBDA_COE / (N * (N - 1) / 2)

        o0 = o0_ref[...]
        o1 = o1_ref[...]
        lab = lab_ref[...]
        m = jnp.maximum(o0, o1)
        lse = m + jnp.log(jnp.exp(o0 - m) + jnp.exp(o1 - m))
        chosen = jnp.where(lab == 1, o1, o0)
        ce = jnp.sum(lse - chosen) / B

        out_ref[...] = jnp.reshape(ce + reg, (1, 1))


def kernel(outputs, labels, H):
    o0 = outputs[:, 0].reshape(64, 128)
    o1 = outputs[:, 1].reshape(64, 128)
    lab = labels.astype(jnp.int32).reshape(64, 128)

    out = pl.pallas_call(
        _body,
        grid=(NBLK,),
        in_specs=[
            pl.BlockSpec((64, 128), lambda i: (0, 0)),
            pl.BlockSpec((64, 128), lambda i: (0, 0)),
            pl.BlockSpec((64, 128), lambda i: (0, 0)),
            pl.BlockSpec((D, BN), lambda i: (0, i)),
        ],
        out_specs=pl.BlockSpec((1, 1), lambda i: (0, 0)),
        out_shape=jax.ShapeDtypeStruct((1, 1), jnp.float32),
        scratch_shapes=[
            pltpu.VMEM((1, D), jnp.float32),
            pltpu.VMEM((1, BN), jnp.float32),
        ],
        compiler_params=pltpu.CompilerParams(
            dimension_semantics=("arbitrary",),
        ),
    )(o0, o1, lab, H)
    return out[0, 0]
